# baseline (speedup 1.0000x reference)
import functools

import jax
import jax.numpy as jnp
import numpy as np
from jax.experimental import pallas as pl
from jax.experimental.pallas import tpu as pltpu
from jax.sharding import Mesh, PartitionSpec as P

import inspect as _inspect

try:
    from jax.experimental.shard_map import shard_map as _smap
except (ImportError, AttributeError):
    _smap = jax.shard_map
_smap_params = _inspect.signature(_smap).parameters
if "check_rep" in _smap_params:
    _shard_map = functools.partial(_smap, check_rep=False)
elif "check_vma" in _smap_params:
    _shard_map = functools.partial(_smap, check_vma=False)
else:
    _shard_map = _smap

_N = 64
_H = 128
_W = 128


def _conv3x3(x, h, w, c, wd, b, *, pool):
    hw = h * w
    x3 = x.reshape(h, w, c)
    zc = jnp.zeros((h, 1, c), jnp.bfloat16)
    wide = jnp.concatenate([
        jnp.concatenate([zc, x3[:, :w - 1]], axis=1),
        x3,
        jnp.concatenate([x3[:, 1:], zc], axis=1)], axis=2)
    w2d = wide.reshape(hw, 3 * c)
    zr = jnp.zeros((w, 3 * c), jnp.bfloat16)
    wp = jnp.concatenate([zr, w2d, zr], axis=0)
    acc = jnp.dot(wp[0:hw], wd[0], preferred_element_type=jnp.float32)
    acc = acc + jnp.dot(wp[w:w + hw], wd[1], preferred_element_type=jnp.float32)
    acc = acc + jnp.dot(wp[2 * w:2 * w + hw], wd[2], preferred_element_type=jnp.float32)
    if pool:
        cout = acc.shape[-1]
        a3 = acc.reshape(hw // 2, 2, cout)
        aw = jnp.maximum(a3[:, 0, :], a3[:, 1, :])
        a4 = aw.reshape(h // 2, 2, w // 2, cout)
        ah = jnp.maximum(a4[:, 0, :, :], a4[:, 1, :, :])
        acc = ah.reshape(hw // 4, cout)
    return jnp.maximum(acc + b, 0.0).astype(jnp.bfloat16)


def _l1_im2col_T(xc):
    xb = xc.astype(jnp.bfloat16)
    hw = xb.shape[1]
    wpos = jax.lax.broadcasted_iota(jnp.int32, xb.shape, 1) % _W
    not_first = wpos != 0
    not_last = wpos != _W - 1
    zrow = jnp.zeros((3, _W), jnp.bfloat16)
    z1 = jnp.zeros((3, 1), jnp.bfloat16)
    taps = []
    for xs in (jnp.concatenate([zrow, xb[:, :hw - _W]], axis=1),
               xb,
               jnp.concatenate([xb[:, _W:], zrow], axis=1)):
        left = jnp.where(not_first,
                         jnp.concatenate([z1, xs[:, :hw - 1]], axis=1), 0)
        right = jnp.where(not_last,
                          jnp.concatenate([xs[:, 1:], z1], axis=1), 0)
        taps += [left, xs, right]
    return jnp.concatenate(taps, axis=0)


def _conv3x3_cmajor(x, h, w, c, wdT, b, *, pool):
    hw = h * w
    xT = x if x.shape[0] == c else jnp.transpose(x)
    wpos = jax.lax.broadcasted_iota(jnp.int32, (c, hw), 1) % w
    not_first = wpos != 0
    not_last = wpos != w - 1
    zrow = jnp.zeros((c, w), jnp.bfloat16)
    z1 = jnp.zeros((c, 1), jnp.bfloat16)
    taps = []
    for xs in (jnp.concatenate([zrow, xT[:, :hw - w]], axis=1),
               xT,
               jnp.concatenate([xT[:, w:], zrow], axis=1)):
        left = jnp.where(not_first,
                         jnp.concatenate([z1, xs[:, :hw - 1]], axis=1), 0)
        right = jnp.where(not_last,
                          jnp.concatenate([xs[:, 1:], z1], axis=1), 0)
        taps += [left, xs, right]
    rowsT = jnp.concatenate(taps, axis=0)
    accT = jnp.dot(wdT, rowsT, preferred_element_type=jnp.float32)
    acc = jnp.transpose(accT)
    if pool:
        cout = acc.shape[-1]
        a3 = acc.reshape(hw // 2, 2, cout)
        aw = jnp.maximum(a3[:, 0, :], a3[:, 1, :])
        a4 = aw.reshape(h // 2, 2, w // 2, cout)
        ah = jnp.maximum(a4[:, 0, :, :], a4[:, 1, :, :])
        acc = ah.reshape(hw // 4, cout)
    return jnp.maximum(acc + b, 0.0).astype(jnp.bfloat16)


def _loss_kernel(fc_ref, rc_ref, fo_ref,
                 w1_ref, b1_ref, w2_ref, b2_ref, w3_ref, b3_ref,
                 w4_ref, b4_ref, w5_ref, b5_ref,
                 o_pix_ref, o_feat_ref, o_fo_ref):
    def l1(rowsT):
        accT = jnp.dot(w1_ref[...], rowsT, preferred_element_type=jnp.float32)
        return jnp.maximum(accT + b1_ref[...], 0.0).astype(jnp.bfloat16)

    layers = [
        (_H // 2, _W // 2, 64, w3_ref, b3_ref, False),
        (_H // 2, _W // 2, 128, w4_ref, b4_ref, True),
        (_H // 4, _W // 4, 128, w5_ref, b5_ref, False),
    ]
    ff = l1(_l1_im2col_T(fc_ref[0]))
    fr = l1(_l1_im2col_T(rc_ref[0]))
    ff = _conv3x3_cmajor(ff, _H, _W, 64, w2_ref[...], b2_ref[...], pool=True)
    fr = _conv3x3_cmajor(fr, _H, _W, 64, w2_ref[...], b2_ref[...], pool=True)
    for (h, w, c, w_ref, b_ref, pool) in layers:
        ff = _conv3x3(ff, h, w, c, w_ref[...], b_ref[...], pool=pool)
        fr = _conv3x3(fr, h, w, c, w_ref[...], b_ref[...], pool=pool)
    df = ff.astype(jnp.float32) - fr.astype(jnp.float32)
    o_feat_ref[0] = jnp.sum(df * df, axis=0, keepdims=True)

    dp = fc_ref[0] - rc_ref[0]
    o_pix_ref[0] = jnp.sum(dp * dp, axis=0, keepdims=True)

    o_fo_ref[0] = jnp.sum(fo_ref[...], axis=0, keepdims=True)


def _pallas_losses(fc, rc, fo, w1r, b1r, w2r, b2r, w3r, b3r, w4r, b4r, w5r, b5r):
    n, _, hw = fc.shape
    return pl.pallas_call(
        _loss_kernel,
        out_shape=[jax.ShapeDtypeStruct((n, 1, hw), jnp.float32),
                   jax.ShapeDtypeStruct((n, 1, 256), jnp.float32),
                   jax.ShapeDtypeStruct((n, 1, 128), jnp.float32)],
        grid=(n,),
        in_specs=[
            pl.BlockSpec((1, 3, hw), lambda i: (i, 0, 0)),
            pl.BlockSpec((1, 3, hw), lambda i: (i, 0, 0)),
            pl.BlockSpec((8, 128), lambda i: (0, 0)),
            pl.BlockSpec((64, 27), lambda i: (0, 0)),
            pl.BlockSpec((64, 1), lambda i: (0, 0)),
            pl.BlockSpec((64, 576), lambda i: (0, 0)),
            pl.BlockSpec((1, 64), lambda i: (0, 0)),
            pl.BlockSpec((3, 192, 128), lambda i: (0, 0, 0)),
            pl.BlockSpec((1, 128), lambda i: (0, 0)),
            pl.BlockSpec((3, 384, 128), lambda i: (0, 0, 0)),
            pl.BlockSpec((1, 128), lambda i: (0, 0)),
            pl.BlockSpec((3, 384, 256), lambda i: (0, 0, 0)),
            pl.BlockSpec((1, 256), lambda i: (0, 0)),
        ],
        out_specs=[pl.BlockSpec((1, 1, hw), lambda i: (i, 0, 0)),
                   pl.BlockSpec((1, 1, 256), lambda i: (i, 0, 0)),
                   pl.BlockSpec((1, 1, 128), lambda i: (i, 0, 0))],
        compiler_params=pltpu.CompilerParams(
            dimension_semantics=("arbitrary",),
            vmem_limit_bytes=100 * 1024 * 1024),
    )(fc, rc, fo,
      w1r, b1r, w2r, b2r, w3r, b3r, w4r, b4r, w5r, b5r)


@jax.jit
def _run(fake_out, fake, real, w0, b0, w1, b1, w2, b2, w3, b3, w4, b4):
    n, _, h, w = fake.shape
    hw = h * w
    fc = fake.reshape(n, 3, hw)
    rc = real.reshape(n, 3, hw)
    fo = jnp.pad(fake_out.reshape(-1), (0, 8 * 128 - n)).reshape(8, 128)

    bf = jnp.bfloat16
    w1r = jnp.transpose(w0, (3, 0, 1, 2)).reshape(64, 27).astype(bf)
    w2r = jnp.transpose(w1, (3, 0, 1, 2)).reshape(64, 576).astype(bf)
    w3r = w2.reshape(3, 192, 128).astype(bf)
    w4r = w3.reshape(3, 384, 128).astype(bf)
    w5r = w4.reshape(3, 384, 256).astype(bf)
    b1r = b0.reshape(64, 1)
    b2r = b1.reshape(1, 64)
    b3r = b2.reshape(1, 128)
    b4r = b3.reshape(1, 128)
    b5r = b4.reshape(1, 256)

    devs = jax.devices()
    nd = len(devs) if n % max(len(devs), 1) == 0 else 1
    mesh = Mesh(np.array(devs[:nd]), ("b",))
    rep = P(*(None,) * 2)
    rep3 = P(*(None,) * 3)
    o_pix, o_feat, o_fo = _shard_map(
        _pallas_losses, mesh=mesh,
        in_specs=(P("b", None, None), P("b", None, None), rep,
                  rep, rep, rep, rep, rep3, rep, rep3, rep, rep3, rep),
        out_specs=(P("b", None, None), P("b", None, None), P("b", None, None)),
    )(fc, rc, fo, w1r, b1r, w2r, b2r, w3r, b3r, w4r, b4r, w5r, b5r)

    n_pix = float(n * 3 * h * w)
    n_feat = float(n * (h // 4) * (w // 4) * 256)
    mse_pix = jnp.sum(o_pix) / n_pix
    mse_feat = jnp.sum(o_feat) / n_feat
    adv = 1.0 - jnp.sum(o_fo[0]) / float(n)
    return mse_pix + 0.006 * mse_feat + 0.001 * adv


def kernel(fake_out, fake, real, w0, b0, w1, b1, w2, b2, w3, b3, w4, b4):
    return _run(fake_out, fake, real, w0, b0, w1, b1, w2, b2, w3, b3, w4, b4)


# speedup vs baseline: 1.0199x; 1.0199x over previous
import functools

import jax
import jax.numpy as jnp
import numpy as np
from jax.experimental import pallas as pl
from jax.experimental.pallas import tpu as pltpu
from jax.sharding import Mesh, PartitionSpec as P

import inspect as _inspect

try:
    from jax.experimental.shard_map import shard_map as _smap
except (ImportError, AttributeError):
    _smap = jax.shard_map
_smap_params = _inspect.signature(_smap).parameters
if "check_rep" in _smap_params:
    _shard_map = functools.partial(_smap, check_rep=False)
elif "check_vma" in _smap_params:
    _shard_map = functools.partial(_smap, check_vma=False)
else:
    _shard_map = _smap

_H = 128
_W = 128


def _conv3x3(x, h, w, c, wd, b, *, pool):
    hw = h * w
    x3 = x.reshape(h, w, c)
    zc = jnp.zeros((h, 1, c), jnp.bfloat16)
    wide = jnp.concatenate([
        jnp.concatenate([zc, x3[:, :w - 1]], axis=1),
        x3,
        jnp.concatenate([x3[:, 1:], zc], axis=1)], axis=2)
    w2d = wide.reshape(hw, 3 * c)
    zr = jnp.zeros((w, 3 * c), jnp.bfloat16)
    wp = jnp.concatenate([zr, w2d, zr], axis=0)
    acc = jnp.dot(wp[0:hw], wd[0], preferred_element_type=jnp.float32)
    acc = acc + jnp.dot(wp[w:w + hw], wd[1], preferred_element_type=jnp.float32)
    acc = acc + jnp.dot(wp[2 * w:2 * w + hw], wd[2], preferred_element_type=jnp.float32)
    if pool:
        cout = acc.shape[-1]
        a3 = acc.reshape(hw // 2, 2, cout)
        aw = jnp.maximum(a3[:, 0, :], a3[:, 1, :])
        a4 = aw.reshape(h // 2, 2, w // 2, cout)
        ah = jnp.maximum(a4[:, 0, :, :], a4[:, 1, :, :])
        acc = ah.reshape(hw // 4, cout)
    return jnp.maximum(acc + b, 0.0).astype(jnp.bfloat16)


def _l1_im2col_T(xc):
    xb = xc.astype(jnp.bfloat16)
    hw = xb.shape[1]
    wpos = jax.lax.broadcasted_iota(jnp.int32, xb.shape, 1) % _W
    not_first = wpos != 0
    not_last = wpos != _W - 1
    zrow = jnp.zeros((3, _W), jnp.bfloat16)
    z1 = jnp.zeros((3, 1), jnp.bfloat16)
    taps = []
    for xs in (jnp.concatenate([zrow, xb[:, :hw - _W]], axis=1),
               xb,
               jnp.concatenate([xb[:, _W:], zrow], axis=1)):
        left = jnp.where(not_first,
                         jnp.concatenate([z1, xs[:, :hw - 1]], axis=1), 0)
        right = jnp.where(not_last,
                          jnp.concatenate([xs[:, 1:], z1], axis=1), 0)
        taps += [left, xs, right]
    return jnp.concatenate(taps, axis=0)


def _conv3x3_cmajor(x, h, w, c, wdT, b, *, pool):
    hw = h * w
    xT = x if x.shape[0] == c else jnp.transpose(x)
    wpos = jax.lax.broadcasted_iota(jnp.int32, (c, hw), 1) % w
    not_first = wpos != 0
    not_last = wpos != w - 1
    zrow = jnp.zeros((c, w), jnp.bfloat16)
    z1 = jnp.zeros((c, 1), jnp.bfloat16)
    taps = []
    for xs in (jnp.concatenate([zrow, xT[:, :hw - w]], axis=1),
               xT,
               jnp.concatenate([xT[:, w:], zrow], axis=1)):
        left = jnp.where(not_first,
                         jnp.concatenate([z1, xs[:, :hw - 1]], axis=1), 0)
        right = jnp.where(not_last,
                          jnp.concatenate([xs[:, 1:], z1], axis=1), 0)
        taps += [left, xs, right]
    rowsT = jnp.concatenate(taps, axis=0)
    accT = jnp.dot(wdT, rowsT, preferred_element_type=jnp.float32)
    acc = jnp.transpose(accT)
    if pool:
        cout = acc.shape[-1]
        a3 = acc.reshape(hw // 2, 2, cout)
        aw = jnp.maximum(a3[:, 0, :], a3[:, 1, :])
        a4 = aw.reshape(h // 2, 2, w // 2, cout)
        ah = jnp.maximum(a4[:, 0, :, :], a4[:, 1, :, :])
        acc = ah.reshape(hw // 4, cout)
    return jnp.maximum(acc + b, 0.0).astype(jnp.bfloat16)


def _loss_kernel(fc_ref, rc_ref, fo_ref,
                 w1_ref, b1_ref, w2_ref, b2_ref, w3_ref, b3_ref,
                 w4_ref, b4_ref, w5_ref, b5_ref,
                 o_pix_ref, o_feat_ref, o_fo_ref):
    def l1(rowsT):
        accT = jnp.dot(w1_ref[...], rowsT, preferred_element_type=jnp.float32)
        return jnp.maximum(accT + b1_ref[...], 0.0).astype(jnp.bfloat16)

    layers = [
        (_H // 2, _W // 2, 64, w3_ref, b3_ref, False),
        (_H // 2, _W // 2, 128, w4_ref, b4_ref, True),
        (_H // 4, _W // 4, 128, w5_ref, b5_ref, False),
    ]
    ff = l1(_l1_im2col_T(fc_ref[0]))
    fr = l1(_l1_im2col_T(rc_ref[0]))
    ff = _conv3x3_cmajor(ff, _H, _W, 64, w2_ref[...], b2_ref[...], pool=True)
    fr = _conv3x3_cmajor(fr, _H, _W, 64, w2_ref[...], b2_ref[...], pool=True)
    for (h, w, c, w_ref, b_ref, pool) in layers:
        ff = _conv3x3(ff, h, w, c, w_ref[...], b_ref[...], pool=pool)
        fr = _conv3x3(fr, h, w, c, w_ref[...], b_ref[...], pool=pool)
    df = ff.astype(jnp.float32) - fr.astype(jnp.float32)
    o_feat_ref[0] = jnp.sum(df * df, axis=0, keepdims=True)

    dp = fc_ref[0] - rc_ref[0]
    o_pix_ref[0] = jnp.sum(dp * dp, axis=0, keepdims=True)

    o_fo_ref[0] = jnp.sum(fo_ref[...], axis=0, keepdims=True)


def _pallas_losses(fc, rc, fo, w1r, b1r, w2r, b2r, w3r, b3r, w4r, b4r, w5r, b5r):
    n, _, hw = fc.shape
    return pl.pallas_call(
        _loss_kernel,
        out_shape=[jax.ShapeDtypeStruct((n, 1, hw), jnp.float32),
                   jax.ShapeDtypeStruct((n, 1, 256), jnp.float32),
                   jax.ShapeDtypeStruct((n, 1, 128), jnp.float32)],
        grid=(n,),
        in_specs=[
            pl.BlockSpec((1, 3, hw), lambda i: (i, 0, 0)),
            pl.BlockSpec((1, 3, hw), lambda i: (i, 0, 0)),
            pl.BlockSpec((8, 128), lambda i: (0, 0)),
            pl.BlockSpec((64, 27), lambda i: (0, 0)),
            pl.BlockSpec((64, 1), lambda i: (0, 0)),
            pl.BlockSpec((64, 576), lambda i: (0, 0)),
            pl.BlockSpec((1, 64), lambda i: (0, 0)),
            pl.BlockSpec((3, 192, 128), lambda i: (0, 0, 0)),
            pl.BlockSpec((1, 128), lambda i: (0, 0)),
            pl.BlockSpec((3, 384, 128), lambda i: (0, 0, 0)),
            pl.BlockSpec((1, 128), lambda i: (0, 0)),
            pl.BlockSpec((3, 384, 256), lambda i: (0, 0, 0)),
            pl.BlockSpec((1, 256), lambda i: (0, 0)),
        ],
        out_specs=[pl.BlockSpec((1, 1, hw), lambda i: (i, 0, 0)),
                   pl.BlockSpec((1, 1, 256), lambda i: (i, 0, 0)),
                   pl.BlockSpec((1, 1, 128), lambda i: (i, 0, 0))],
        compiler_params=pltpu.CompilerParams(
            dimension_semantics=("arbitrary",),
            vmem_limit_bytes=64 * 1024 * 1024),
    )(fc, rc, fo,
      w1r, b1r, w2r, b2r, w3r, b3r, w4r, b4r, w5r, b5r)


@jax.jit
def _run(fake_out, fake, real, w0, b0, w1, b1, w2, b2, w3, b3, w4, b4):
    n, _, h, w = fake.shape
    hw = h * w
    fc = fake.reshape(n, 3, hw)
    rc = real.reshape(n, 3, hw)
    fo = jnp.pad(fake_out.reshape(-1), (0, 8 * 128 - n)).reshape(8, 128)

    bf = jnp.bfloat16
    w1r = jnp.transpose(w0, (3, 0, 1, 2)).reshape(64, 27).astype(bf)
    w2r = jnp.transpose(w1, (3, 0, 1, 2)).reshape(64, 576).astype(bf)
    w3r = w2.reshape(3, 192, 128).astype(bf)
    w4r = w3.reshape(3, 384, 128).astype(bf)
    w5r = w4.reshape(3, 384, 256).astype(bf)
    b1r = b0.reshape(64, 1)
    b2r = b1.reshape(1, 64)
    b3r = b2.reshape(1, 128)
    b4r = b3.reshape(1, 128)
    b5r = b4.reshape(1, 256)

    devs = jax.devices()
    nd = len(devs) if n % max(len(devs), 1) == 0 else 1
    mesh = Mesh(np.array(devs[:nd]), ("b",))
    rep = P(*(None,) * 2)
    rep3 = P(*(None,) * 3)
    o_pix, o_feat, o_fo = _shard_map(
        _pallas_losses, mesh=mesh,
        in_specs=(P("b", None, None), P("b", None, None), rep,
                  rep, rep, rep, rep, rep3, rep, rep3, rep, rep3, rep),
        out_specs=(P("b", None, None), P("b", None, None), P("b", None, None)),
    )(fc, rc, fo, w1r, b1r, w2r, b2r, w3r, b3r, w4r, b4r, w5r, b5r)

    n_pix = float(n * 3 * h * w)
    n_feat = float(n * (h // 4) * (w // 4) * 256)
    mse_pix = jnp.sum(o_pix) / n_pix
    mse_feat = jnp.sum(o_feat) / n_feat
    adv = 1.0 - jnp.sum(o_fo[0]) / float(n)
    return mse_pix + 0.006 * mse_feat + 0.001 * adv


def kernel(fake_out, fake, real, w0, b0, w1, b1, w2, b2, w3, b3, w4, b4):
    return _run(fake_out, fake, real, w0, b0, w1, b1, w2, b2, w3, b3, w4, b4)
